# Initial kernel scaffold; baseline (speedup 1.0000x reference)
import functools

import jax
import jax.numpy as jnp
from jax.experimental import pallas as pl
from jax.experimental.pallas import tpu as pltpu

L = 2
E = 8
H = 768
NH = 12
DH = 64
FF = 3072
V = 30522
B = 1
S = 2048

_BF = jnp.bfloat16
_F32 = jnp.float32


def _ln(x, g, b):
    m = jnp.mean(x, -1, keepdims=True)
    v = jnp.mean((x - m) ** 2, -1, keepdims=True)
    return (x - m) / jnp.sqrt(v + 1e-12) * g + b


def _router_kernel(hs_ref, cc_ref, eid_ref):
    for l in range(L):
        logits = jnp.sum(cc_ref[l] * hs_ref[l], axis=-1)
        eid_ref[l, 0] = jnp.argmax(logits).astype(jnp.int32)


def _route(hidden_states_for_router, cluster_centers):
    return pl.pallas_call(
        _router_kernel,
        out_shape=jax.ShapeDtypeStruct((L, B), jnp.int32),
        in_specs=[
            pl.BlockSpec(memory_space=pltpu.VMEM),
            pl.BlockSpec(memory_space=pltpu.VMEM),
        ],
        out_specs=pl.BlockSpec(memory_space=pltpu.SMEM),
    )(hidden_states_for_router, cluster_centers)


_ER = 16


def _embed_kernel(ids_ref, *refs):
    del ids_ref
    we = refs[:_ER]
    pos_ref, g_ref, b_ref, out_ref = refs[_ER:]
    x = jnp.concatenate([r[...] for r in we], axis=0) + pos_ref[...]
    out_ref[...] = _ln(x, g_ref[...], b_ref[...])


def _embed(input_ids, word_emb, pos_emb, g, b):
    ids = input_ids.reshape(S).astype(jnp.int32)
    grid = S // _ER
    we_specs = [
        pl.BlockSpec((1, H), functools.partial(lambda j, i, ids: (ids[i * _ER + j], 0), j))
        for j in range(_ER)
    ]
    out = pl.pallas_call(
        _embed_kernel,
        grid_spec=pltpu.PrefetchScalarGridSpec(
            num_scalar_prefetch=1,
            grid=(grid,),
            in_specs=we_specs + [
                pl.BlockSpec((_ER, H), lambda i, ids: (i, 0)),
                pl.BlockSpec((1, H), lambda i, ids: (0, 0)),
                pl.BlockSpec((1, H), lambda i, ids: (0, 0)),
            ],
            out_specs=pl.BlockSpec((_ER, H), lambda i, ids: (i, 0)),
        ),
        out_shape=jax.ShapeDtypeStruct((S, H), _F32),
    )(ids, *([word_emb] * _ER), pos_emb, g.reshape(1, H), b.reshape(1, H))
    return out


_AT = 256


def _attn_kernel(h_ref, mask_ref, wq_ref, bq_ref, wk_ref, bk_ref, wv_ref,
                 bv_ref, wo_ref, bo_ref, g_ref, b_ref, out_ref,
                 k_scr, v_scr):
    i = pl.program_id(0)

    @pl.when(i == 0)
    def _():
        hb = h_ref[...].astype(_BF)
        k = jnp.dot(hb, wk_ref[...].astype(_BF), preferred_element_type=_F32) + bk_ref[...]
        v = jnp.dot(hb, wv_ref[...].astype(_BF), preferred_element_type=_F32) + bv_ref[...]
        k_scr[...] = k.astype(_BF)
        v_scr[...] = v.astype(_BF)

    h_tile = h_ref[pl.ds(i * _AT, _AT), :]
    q = jnp.dot(h_tile.astype(_BF), wq_ref[...].astype(_BF),
                preferred_element_type=_F32) + bq_ref[...]
    qb = q.astype(_BF)
    neg = (1.0 - mask_ref[...]) * -1e9
    scale = 1.0 / (DH ** 0.5)
    ctx = []
    for hh in range(NH):
        sl = slice(hh * DH, (hh + 1) * DH)
        q_h = qb[:, sl]
        k_h = k_scr[:, sl]
        v_h = v_scr[:, sl]
        s = jax.lax.dot_general(q_h, k_h, (((1,), (1,)), ((), ())),
                                preferred_element_type=_F32)
        s = s * scale + neg
        s = s - jnp.max(s, axis=-1, keepdims=True)
        e = jnp.exp(s)
        p = e / jnp.sum(e, axis=-1, keepdims=True)
        ctx.append(jnp.dot(p.astype(_BF), v_h, preferred_element_type=_F32))
    c = jnp.concatenate(ctx, axis=1)
    o = jnp.dot(c.astype(_BF), wo_ref[...].astype(_BF),
                preferred_element_type=_F32) + bo_ref[...]
    out_ref[...] = _ln(h_tile + o, g_ref[...], b_ref[...])


def _attention(h, mask, lp):
    row = lambda x: x.reshape(1, H)
    full = pl.BlockSpec((S, H), lambda i: (0, 0))
    wspec = pl.BlockSpec((H, H), lambda i: (0, 0))
    bspec = pl.BlockSpec((1, H), lambda i: (0, 0))
    return pl.pallas_call(
        _attn_kernel,
        grid=(S // _AT,),
        in_specs=[full, pl.BlockSpec((1, S), lambda i: (0, 0)),
                  wspec, bspec, wspec, bspec, wspec, bspec, wspec, bspec,
                  bspec, bspec],
        out_specs=pl.BlockSpec((_AT, H), lambda i: (i, 0)),
        out_shape=jax.ShapeDtypeStruct((S, H), _F32),
        scratch_shapes=[pltpu.VMEM((S, H), _BF), pltpu.VMEM((S, H), _BF)],
        compiler_params=pltpu.CompilerParams(vmem_limit_bytes=100 * 1024 * 1024),
    )(h, mask.reshape(1, S), lp['Wq'], row(lp['bq']), lp['Wk'], row(lp['bk']),
      lp['Wv'], row(lp['bv']), lp['Wo'], row(lp['bo']),
      row(lp['ln1_g']), row(lp['ln1_b']))


_FT = 256


def _ffn_kernel(eid_ref, x_ref, w1_ref, b1_ref, w2_ref, b2_ref, g_ref, b_ref,
                out_ref):
    del eid_ref
    x = x_ref[...]
    f = jnp.dot(x.astype(_BF), w1_ref[0].astype(_BF),
                preferred_element_type=_F32) + b1_ref[0]
    f = jax.nn.gelu(f)
    y = jnp.dot(f.astype(_BF), w2_ref[0].astype(_BF),
                preferred_element_type=_F32) + b2_ref[0]
    out_ref[...] = _ln(x + y, g_ref[...], b_ref[...])


def _ffn(h, eid, lp):
    row = lambda x: x.reshape(1, H)
    return pl.pallas_call(
        _ffn_kernel,
        grid_spec=pltpu.PrefetchScalarGridSpec(
            num_scalar_prefetch=1,
            grid=(S // _FT,),
            in_specs=[
                pl.BlockSpec((_FT, H), lambda i, e: (i, 0)),
                pl.BlockSpec((1, H, FF), lambda i, e: (e[0], 0, 0)),
                pl.BlockSpec((1, 1, FF), lambda i, e: (e[0], 0, 0)),
                pl.BlockSpec((1, FF, H), lambda i, e: (e[0], 0, 0)),
                pl.BlockSpec((1, 1, H), lambda i, e: (e[0], 0, 0)),
                pl.BlockSpec((1, H), lambda i, e: (0, 0)),
                pl.BlockSpec((1, H), lambda i, e: (0, 0)),
            ],
            out_specs=pl.BlockSpec((_FT, H), lambda i, e: (i, 0)),
        ),
        out_shape=jax.ShapeDtypeStruct((S, H), _F32),
        compiler_params=pltpu.CompilerParams(vmem_limit_bytes=100 * 1024 * 1024),
    )(eid, h, lp['W1'], lp['b1'].reshape(E, 1, FF), lp['W2'],
      lp['b2'].reshape(E, 1, H), row(lp['ln2_g']), row(lp['ln2_b']))


def _head_kernel(x_ref, w_ref, b_ref, g_ref, bb_ref, out_ref):
    x = x_ref[...]
    d = jnp.dot(x.astype(_BF), w_ref[...].astype(_BF),
                preferred_element_type=_F32) + b_ref[...]
    d = _ln(jax.nn.gelu(d), g_ref[...], bb_ref[...])
    out_ref[...] = d.astype(_BF)


def _head(h, hd):
    row = lambda x: x.reshape(1, H)
    bspec = pl.BlockSpec((1, H), lambda i: (0, 0))
    return pl.pallas_call(
        _head_kernel,
        grid=(S // _FT,),
        in_specs=[pl.BlockSpec((_FT, H), lambda i: (i, 0)),
                  pl.BlockSpec((H, H), lambda i: (0, 0)), bspec, bspec, bspec],
        out_specs=pl.BlockSpec((_FT, H), lambda i: (i, 0)),
        out_shape=jax.ShapeDtypeStruct((S, H), _BF),
    )(h, hd['Wd'], row(hd['bd']), row(hd['ln_g']), row(hd['ln_b']))


_VT = 1024
_NV = (V + _VT - 1) // _VT


def _dec_kernel(d_ref, w_ref, bd_ref, lab_ref, scores_ref, loss_ref,
                m_scr, l_scr, p_scr):
    j = pl.program_id(0)
    w = w_ref[...].astype(_BF)
    s = jnp.dot(d_ref[...], w, preferred_element_type=_F32) + bd_ref[...]
    scores_ref[...] = s
    col = j * _VT + jax.lax.broadcasted_iota(jnp.int32, (1, _VT), 1)
    valid = col < V
    sm = jnp.where(valid, s, -1e30)
    tile_max = jnp.max(sm, axis=-1, keepdims=True)
    hit = col == lab_ref[...]
    p_add = jnp.sum(jnp.where(hit, s, 0.0), axis=-1, keepdims=True)

    @pl.when(j == 0)
    def _():
        m_scr[...] = jnp.full((S, 1), -1e30, _F32)
        l_scr[...] = jnp.zeros((S, 1), _F32)
        p_scr[...] = jnp.zeros((S, 1), _F32)

    m_old = m_scr[...]
    m_new = jnp.maximum(m_old, tile_max)
    l_scr[...] = (l_scr[...] * jnp.exp(m_old - m_new)
                  + jnp.sum(jnp.exp(sm - m_new), axis=-1, keepdims=True))
    m_scr[...] = m_new
    p_scr[...] = p_scr[...] + p_add

    @pl.when(j == _NV - 1)
    def _():
        lse = m_scr[...] + jnp.log(l_scr[...])
        nll = lse - p_scr[...]
        loss_ref[...] = jnp.sum(nll, axis=0, keepdims=True) / (B * S)


def _decode(d, labels, hd):
    scores, loss = pl.pallas_call(
        _dec_kernel,
        grid=(_NV,),
        in_specs=[
            pl.BlockSpec((S, H), lambda j: (0, 0)),
            pl.BlockSpec((H, _VT), lambda j: (0, j)),
            pl.BlockSpec((1, _VT), lambda j: (0, j)),
            pl.BlockSpec((S, 1), lambda j: (0, 0)),
        ],
        out_specs=[
            pl.BlockSpec((S, _VT), lambda j: (0, j)),
            pl.BlockSpec((1, 1), lambda j: (0, 0)),
        ],
        out_shape=[
            jax.ShapeDtypeStruct((S, V), _F32),
            jax.ShapeDtypeStruct((1, 1), _F32),
        ],
        scratch_shapes=[pltpu.VMEM((S, 1), _F32)] * 3,
        compiler_params=pltpu.CompilerParams(vmem_limit_bytes=100 * 1024 * 1024),
    )(d, hd['Wdec'], hd['bdec'].reshape(1, V),
      labels.reshape(S, 1).astype(jnp.int32))
    return scores, loss


def kernel(input_ids, attention_mask, labels, cluster_centers,
           hidden_states_for_router, params):
    eids = _route(hidden_states_for_router, cluster_centers)
    h = _embed(input_ids, params['word_emb'], params['pos_emb'],
               params['emb_ln_g'], params['emb_ln_b'])
    mask = attention_mask.astype(_F32)
    for i in range(L):
        lp = params['layers'][i]
        h = _attention(h, mask, lp)
        h = _ffn(h, eids[i], lp)
    d = _head(h, params['head'])
    scores, loss = _decode(d, labels, params['head'])
    return loss[0, 0], scores.reshape(B, S, V), eids


# speedup vs baseline: 1.1693x; 1.1693x over previous
import functools

import jax
import jax.numpy as jnp
from jax.experimental import pallas as pl
from jax.experimental.pallas import tpu as pltpu

L = 2
E = 8
H = 768
NH = 12
DH = 64
FF = 3072
V = 30522
B = 1
S = 2048

_BF = jnp.bfloat16
_F32 = jnp.float32


def _ln(x, g, b):
    m = jnp.mean(x, -1, keepdims=True)
    v = jnp.mean((x - m) ** 2, -1, keepdims=True)
    return (x - m) / jnp.sqrt(v + 1e-12) * g + b


def _router_kernel(hs_ref, cc_ref, eid_ref):
    for l in range(L):
        logits = jnp.sum(cc_ref[l] * hs_ref[l], axis=-1)
        eid_ref[l, 0] = jnp.argmax(logits).astype(jnp.int32)


def _route(hidden_states_for_router, cluster_centers):
    return pl.pallas_call(
        _router_kernel,
        out_shape=jax.ShapeDtypeStruct((L, B), jnp.int32),
        in_specs=[
            pl.BlockSpec(memory_space=pltpu.VMEM),
            pl.BlockSpec(memory_space=pltpu.VMEM),
        ],
        out_specs=pl.BlockSpec(memory_space=pltpu.SMEM),
    )(hidden_states_for_router, cluster_centers)


_ER = 16


def _embed_kernel(ids_ref, *refs):
    del ids_ref
    we = refs[:_ER]
    pos_ref, g_ref, b_ref, out_ref = refs[_ER:]
    x = jnp.concatenate([r[0] for r in we], axis=0) + pos_ref[...]
    out_ref[...] = _ln(x, g_ref[...], b_ref[...])


def _embed(input_ids, word_emb, pos_emb, g, b):
    ids = input_ids.reshape(S).astype(jnp.int32)
    grid = S // _ER
    we3 = word_emb.reshape(V, 1, H)
    we_specs = [
        pl.BlockSpec((1, 1, H),
                     functools.partial(lambda j, i, ids: (ids[i * _ER + j], 0, 0), j))
        for j in range(_ER)
    ]
    out = pl.pallas_call(
        _embed_kernel,
        grid_spec=pltpu.PrefetchScalarGridSpec(
            num_scalar_prefetch=1,
            grid=(grid,),
            in_specs=we_specs + [
                pl.BlockSpec((_ER, H), lambda i, ids: (i, 0)),
                pl.BlockSpec((1, H), lambda i, ids: (0, 0)),
                pl.BlockSpec((1, H), lambda i, ids: (0, 0)),
            ],
            out_specs=pl.BlockSpec((_ER, H), lambda i, ids: (i, 0)),
        ),
        out_shape=jax.ShapeDtypeStruct((S, H), _F32),
    )(ids, *([we3] * _ER), pos_emb, g.reshape(1, H), b.reshape(1, H))
    return out


_AT = 256


def _attn_kernel(h_ref, mask_ref, wq_ref, bq_ref, wk_ref, bk_ref, wv_ref,
                 bv_ref, wo_ref, bo_ref, g_ref, b_ref, out_ref,
                 k_scr, v_scr):
    i = pl.program_id(0)

    @pl.when(i == 0)
    def _():
        hb = h_ref[...].astype(_BF)
        k = jnp.dot(hb, wk_ref[...].astype(_BF), preferred_element_type=_F32) + bk_ref[...]
        v = jnp.dot(hb, wv_ref[...].astype(_BF), preferred_element_type=_F32) + bv_ref[...]
        k_scr[...] = k.astype(_BF)
        v_scr[...] = v.astype(_BF)

    h_tile = h_ref[pl.ds(i * _AT, _AT), :]
    q = jnp.dot(h_tile.astype(_BF), wq_ref[...].astype(_BF),
                preferred_element_type=_F32) + bq_ref[...]
    qb = q.astype(_BF)
    neg = (1.0 - mask_ref[...]) * -1e9
    scale = 1.0 / (DH ** 0.5)
    ctx = []
    for hh in range(NH):
        sl = slice(hh * DH, (hh + 1) * DH)
        q_h = qb[:, sl]
        k_h = k_scr[:, sl]
        v_h = v_scr[:, sl]
        s = jax.lax.dot_general(q_h, k_h, (((1,), (1,)), ((), ())),
                                preferred_element_type=_F32)
        s = s * scale + neg
        s = s - jnp.max(s, axis=-1, keepdims=True)
        e = jnp.exp(s)
        p = e / jnp.sum(e, axis=-1, keepdims=True)
        ctx.append(jnp.dot(p.astype(_BF), v_h, preferred_element_type=_F32))
    c = jnp.concatenate(ctx, axis=1)
    o = jnp.dot(c.astype(_BF), wo_ref[...].astype(_BF),
                preferred_element_type=_F32) + bo_ref[...]
    out_ref[...] = _ln(h_tile + o, g_ref[...], b_ref[...])


def _attention(h, mask, lp):
    row = lambda x: x.reshape(1, H)
    full = pl.BlockSpec((S, H), lambda i: (0, 0))
    wspec = pl.BlockSpec((H, H), lambda i: (0, 0))
    bspec = pl.BlockSpec((1, H), lambda i: (0, 0))
    return pl.pallas_call(
        _attn_kernel,
        grid=(S // _AT,),
        in_specs=[full, pl.BlockSpec((1, S), lambda i: (0, 0)),
                  wspec, bspec, wspec, bspec, wspec, bspec, wspec, bspec,
                  bspec, bspec],
        out_specs=pl.BlockSpec((_AT, H), lambda i: (i, 0)),
        out_shape=jax.ShapeDtypeStruct((S, H), _F32),
        scratch_shapes=[pltpu.VMEM((S, H), _BF), pltpu.VMEM((S, H), _BF)],
        compiler_params=pltpu.CompilerParams(vmem_limit_bytes=100 * 1024 * 1024),
    )(h, mask.reshape(1, S), lp['Wq'], row(lp['bq']), lp['Wk'], row(lp['bk']),
      lp['Wv'], row(lp['bv']), lp['Wo'], row(lp['bo']),
      row(lp['ln1_g']), row(lp['ln1_b']))


_FT = 256


def _ffn_kernel(eid_ref, x_ref, w1_ref, b1_ref, w2_ref, b2_ref, g_ref, b_ref,
                out_ref):
    del eid_ref
    x = x_ref[...]
    f = jnp.dot(x.astype(_BF), w1_ref[0].astype(_BF),
                preferred_element_type=_F32) + b1_ref[0]
    f = jax.nn.gelu(f)
    y = jnp.dot(f.astype(_BF), w2_ref[0].astype(_BF),
                preferred_element_type=_F32) + b2_ref[0]
    out_ref[...] = _ln(x + y, g_ref[...], b_ref[...])


def _ffn(h, eid, lp):
    row = lambda x: x.reshape(1, H)
    return pl.pallas_call(
        _ffn_kernel,
        grid_spec=pltpu.PrefetchScalarGridSpec(
            num_scalar_prefetch=1,
            grid=(S // _FT,),
            in_specs=[
                pl.BlockSpec((_FT, H), lambda i, e: (i, 0)),
                pl.BlockSpec((1, H, FF), lambda i, e: (e[0], 0, 0)),
                pl.BlockSpec((1, 1, FF), lambda i, e: (e[0], 0, 0)),
                pl.BlockSpec((1, FF, H), lambda i, e: (e[0], 0, 0)),
                pl.BlockSpec((1, 1, H), lambda i, e: (e[0], 0, 0)),
                pl.BlockSpec((1, H), lambda i, e: (0, 0)),
                pl.BlockSpec((1, H), lambda i, e: (0, 0)),
            ],
            out_specs=pl.BlockSpec((_FT, H), lambda i, e: (i, 0)),
        ),
        out_shape=jax.ShapeDtypeStruct((S, H), _F32),
        compiler_params=pltpu.CompilerParams(vmem_limit_bytes=100 * 1024 * 1024),
    )(eid, h, lp['W1'], lp['b1'].reshape(E, 1, FF), lp['W2'],
      lp['b2'].reshape(E, 1, H), row(lp['ln2_g']), row(lp['ln2_b']))


def _head_kernel(x_ref, w_ref, b_ref, g_ref, bb_ref, out_ref):
    x = x_ref[...]
    d = jnp.dot(x.astype(_BF), w_ref[...].astype(_BF),
                preferred_element_type=_F32) + b_ref[...]
    d = _ln(jax.nn.gelu(d), g_ref[...], bb_ref[...])
    out_ref[...] = d.astype(_BF)


def _head(h, hd):
    row = lambda x: x.reshape(1, H)
    bspec = pl.BlockSpec((1, H), lambda i: (0, 0))
    return pl.pallas_call(
        _head_kernel,
        grid=(S // _FT,),
        in_specs=[pl.BlockSpec((_FT, H), lambda i: (i, 0)),
                  pl.BlockSpec((H, H), lambda i: (0, 0)), bspec, bspec, bspec],
        out_specs=pl.BlockSpec((_FT, H), lambda i: (i, 0)),
        out_shape=jax.ShapeDtypeStruct((S, H), _BF),
    )(h, hd['Wd'], row(hd['bd']), row(hd['ln_g']), row(hd['ln_b']))


_VT = 1024
_NV = (V + _VT - 1) // _VT


def _dec_kernel(d_ref, w_ref, bd_ref, lab_ref, scores_ref, loss_ref,
                m_scr, l_scr, p_scr):
    j = pl.program_id(0)
    w = w_ref[...].astype(_BF)
    s = jnp.dot(d_ref[...], w, preferred_element_type=_F32) + bd_ref[...]
    scores_ref[...] = s
    col = j * _VT + jax.lax.broadcasted_iota(jnp.int32, (1, _VT), 1)
    valid = col < V
    sm = jnp.where(valid, s, -1e30)
    tile_max = jnp.max(sm, axis=-1, keepdims=True)
    hit = col == lab_ref[...]
    p_add = jnp.sum(jnp.where(hit, s, 0.0), axis=-1, keepdims=True)

    @pl.when(j == 0)
    def _():
        m_scr[...] = jnp.full((S, 1), -1e30, _F32)
        l_scr[...] = jnp.zeros((S, 1), _F32)
        p_scr[...] = jnp.zeros((S, 1), _F32)

    m_old = m_scr[...]
    m_new = jnp.maximum(m_old, tile_max)
    l_scr[...] = (l_scr[...] * jnp.exp(m_old - m_new)
                  + jnp.sum(jnp.exp(sm - m_new), axis=-1, keepdims=True))
    m_scr[...] = m_new
    p_scr[...] = p_scr[...] + p_add

    @pl.when(j == _NV - 1)
    def _():
        lse = m_scr[...] + jnp.log(l_scr[...])
        nll = lse - p_scr[...]
        loss_ref[...] = jnp.sum(nll, axis=0, keepdims=True) / (B * S)


def _decode(d, labels, hd):
    scores, loss = pl.pallas_call(
        _dec_kernel,
        grid=(_NV,),
        in_specs=[
            pl.BlockSpec((S, H), lambda j: (0, 0)),
            pl.BlockSpec((H, _VT), lambda j: (0, j)),
            pl.BlockSpec((1, _VT), lambda j: (0, j)),
            pl.BlockSpec((S, 1), lambda j: (0, 0)),
        ],
        out_specs=[
            pl.BlockSpec((S, _VT), lambda j: (0, j)),
            pl.BlockSpec((1, 1), lambda j: (0, 0)),
        ],
        out_shape=[
            jax.ShapeDtypeStruct((S, V), _F32),
            jax.ShapeDtypeStruct((1, 1), _F32),
        ],
        scratch_shapes=[pltpu.VMEM((S, 1), _F32)] * 3,
        compiler_params=pltpu.CompilerParams(vmem_limit_bytes=100 * 1024 * 1024),
    )(d, hd['Wdec'], hd['bdec'].reshape(1, V),
      labels.reshape(S, 1).astype(jnp.int32))
    return scores, loss


def kernel(input_ids, attention_mask, labels, cluster_centers,
           hidden_states_for_router, params):
    eids = _route(hidden_states_for_router, cluster_centers)
    h = _embed(input_ids, params['word_emb'], params['pos_emb'],
               params['emb_ln_g'], params['emb_ln_b'])
    mask = attention_mask.astype(_F32)
    for i in range(L):
        lp = params['layers'][i]
        h = _attention(h, mask, lp)
        h = _ffn(h, eids[i], lp)
    d = _head(h, params['head'])
    scores, loss = _decode(d, labels, params['head'])
    return loss[0, 0], scores.reshape(B, S, V), eids
